# baseline (speedup 1.0000x reference)
import jax
import jax.numpy as jnp
from jax.experimental import pallas as pl

_BLOCK_ROWS = 4000


def _copy_scatter_body(x_ref, o_ref):
    i = pl.program_id(0)

    @pl.when(i == 0)
    def _patch_block():
        blk = x_ref[...]
        r = jax.lax.broadcasted_iota(jnp.int32, blk.shape, 0)
        c = jax.lax.broadcasted_iota(jnp.int32, blk.shape, 1)
        row0 = r == 0
        blk = jnp.where(row0 & (c == 0), jnp.float32(88.0), blk)
        blk = jnp.where(row0 & (c == 64), jnp.float32(99.0), blk)
        o_ref[...] = blk

    @pl.when(i > 0)
    def _copy_block():
        o_ref[...] = x_ref[...]


def kernel(x):
    n, d = x.shape
    xw = x.reshape(n // 2, 2 * d)
    grid = pl.cdiv(n // 2, _BLOCK_ROWS)
    out = pl.pallas_call(
        _copy_scatter_body,
        grid=(grid,),
        in_specs=[pl.BlockSpec((_BLOCK_ROWS, 2 * d), lambda i: (i, 0))],
        out_specs=pl.BlockSpec((_BLOCK_ROWS, 2 * d), lambda i: (i, 0)),
        out_shape=jax.ShapeDtypeStruct((n // 2, 2 * d), x.dtype),
    )(xw)
    return out.reshape(n, d)


# speedup vs baseline: 1.3808x; 1.3808x over previous
import jax
import jax.numpy as jnp
from jax.experimental import pallas as pl

_BLOCK_ROWS = 25000


def _copy_scatter_body(x_ref, o_ref):
    i = pl.program_id(0)

    @pl.when(i == 0)
    def _patch_block():
        blk = x_ref[...]
        r = jax.lax.broadcasted_iota(jnp.int32, blk.shape, 0)
        c = jax.lax.broadcasted_iota(jnp.int32, blk.shape, 1)
        col0 = c == 0
        blk = jnp.where((r == 0) & col0, jnp.float32(88.0), blk)
        blk = jnp.where((r == 1) & col0, jnp.float32(99.0), blk)
        o_ref[...] = blk

    @pl.when(i > 0)
    def _copy_block():
        o_ref[...] = x_ref[...]


def kernel(x):
    n, d = x.shape
    grid = pl.cdiv(n, _BLOCK_ROWS)
    return pl.pallas_call(
        _copy_scatter_body,
        grid=(grid,),
        in_specs=[pl.BlockSpec((_BLOCK_ROWS, d), lambda i: (i, 0))],
        out_specs=pl.BlockSpec((_BLOCK_ROWS, d), lambda i: (i, 0)),
        out_shape=jax.ShapeDtypeStruct((n, d), x.dtype),
    )(x)


# speedup vs baseline: 1.3810x; 1.0001x over previous
import jax
import jax.numpy as jnp
from jax.experimental import pallas as pl

_BLOCK_ROWS = 8000


def _copy_scatter_body(x_ref, o_ref):
    i = pl.program_id(0)

    @pl.when(i == 0)
    def _patch_block():
        blk = x_ref[...]
        r = jax.lax.broadcasted_iota(jnp.int32, blk.shape, 0)
        c = jax.lax.broadcasted_iota(jnp.int32, blk.shape, 1)
        col0 = c == 0
        blk = jnp.where((r == 0) & col0, jnp.float32(88.0), blk)
        blk = jnp.where((r == 1) & col0, jnp.float32(99.0), blk)
        o_ref[...] = blk

    @pl.when(i > 0)
    def _copy_block():
        o_ref[...] = x_ref[...]


def kernel(x):
    n, d = x.shape
    grid = pl.cdiv(n, _BLOCK_ROWS)
    return pl.pallas_call(
        _copy_scatter_body,
        grid=(grid,),
        in_specs=[pl.BlockSpec((_BLOCK_ROWS, d), lambda i: (i, 0))],
        out_specs=pl.BlockSpec((_BLOCK_ROWS, d), lambda i: (i, 0)),
        out_shape=jax.ShapeDtypeStruct((n, d), x.dtype),
    )(x)


# speedup vs baseline: 8.5972x; 6.2255x over previous
import jax
import jax.numpy as jnp
from jax.experimental import pallas as pl

_BLOCK_COLS = 16384


def _copy_scatter_body(x_ref, o_ref):
    i = pl.program_id(0)

    @pl.when(i == 0)
    def _patch_block():
        blk = x_ref[...]
        r = jax.lax.broadcasted_iota(jnp.int32, blk.shape, 0)
        c = jax.lax.broadcasted_iota(jnp.int32, blk.shape, 1)
        row0 = r == 0
        blk = jnp.where(row0 & (c == 0), jnp.float32(88.0), blk)
        blk = jnp.where(row0 & (c == 1), jnp.float32(99.0), blk)
        o_ref[...] = blk

    @pl.when(i > 0)
    def _copy_block():
        o_ref[...] = x_ref[...]


def kernel(x):
    n, d = x.shape
    xt = x.T
    grid = pl.cdiv(n, _BLOCK_COLS)
    out_t = pl.pallas_call(
        _copy_scatter_body,
        grid=(grid,),
        in_specs=[pl.BlockSpec((d, _BLOCK_COLS), lambda i: (0, i))],
        out_specs=pl.BlockSpec((d, _BLOCK_COLS), lambda i: (0, i)),
        out_shape=jax.ShapeDtypeStruct((d, n), x.dtype),
    )(xt)
    return out_t.T
